# Initial kernel scaffold; baseline (speedup 1.0000x reference)
import jax
import jax.numpy as jnp
from jax.experimental import pallas as pl


def kernel(x, edge_index, W1, b1, W2, b2):
    raise NotImplementedError("write your pallas kernel here")


# speedup vs baseline: 88.0541x; 88.0541x over previous
import functools

import jax
import jax.numpy as jnp
from jax import lax
from jax.experimental import pallas as pl
from jax.experimental.pallas import tpu as pltpu
from jax.experimental.pallas import tpu_sc as plsc

NC = 2
NS = 16
L = 128
K = 8

f32 = jnp.float32
i32 = jnp.int32


def _ru(a, b):
    return (a + b - 1) // b * b


def _mesh():
    return plsc.VectorSubcoreMesh(core_axis_name="c", subcore_axis_name="s")


def _make_deg_kernel(n_pad, rows_per_sub):
    S = n_pad // NS

    @functools.partial(
        pl.kernel,
        out_type=jax.ShapeDtypeStruct((NC * n_pad,), f32),
        mesh=_mesh(),
        scratch_types=[
            pltpu.VMEM((K, L), i32),
            pltpu.VMEM((L,), f32),
            pltpu.VMEM((n_pad // NS,), f32),
            pltpu.VMEM_SHARED((n_pad,), f32),
            pltpu.SemaphoreType.DMA,
        ],
    )
    def deg_kernel(dst_hbm, ones_hbm, zeros_hbm, out_hbm, didx, ones_v, bnc, acc, sem):
        c = lax.axis_index("c")
        s = lax.axis_index("s")
        pltpu.sync_copy(ones_hbm.at[pl.ds(0, L)], ones_v)

        @pl.when(c == 0)
        def _():
            pltpu.sync_copy(ones_hbm.at[pl.ds(s * S, S)], bnc)

        @pl.when(c != 0)
        def _():
            pltpu.sync_copy(zeros_hbm.at[pl.ds(s * S, S)], bnc)

        pltpu.sync_copy(bnc, acc.at[pl.ds(s * S, S)])

        plsc.subcore_barrier()
        row_base = (c * NS + s) * jnp.int32(rows_per_sub)

        def body(g, carry):
            pltpu.sync_copy(dst_hbm.at[pl.ds(row_base + g * jnp.int32(K), K)], didx)
            ds = [pltpu.async_copy(ones_v, acc.at[didx.at[jnp.int32(j)]], sem, add=True)
                  for j in range(K)]
            for d in ds:
                d.wait()
            return carry

        lax.fori_loop(jnp.int32(0), jnp.int32(rows_per_sub // K), body, jnp.int32(0))
        plsc.subcore_barrier()
        pltpu.sync_copy(acc.at[pl.ds(s * S, S)], bnc)
        pltpu.sync_copy(bnc, out_hbm.at[pl.ds(c * n_pad + s * S, S)])

    return deg_kernel


def _make_row_agg_kernel(n_pad, rows_per_sub, d):
    S = n_pad // NS

    @functools.partial(
        pl.kernel,
        out_type=jax.ShapeDtypeStruct((NC * n_pad, d), f32),
        mesh=_mesh(),
        compiler_params=pltpu.CompilerParams(use_tc_tiling_on_sc=False),
        scratch_types=[
            pltpu.VMEM((K, L), i32),
            pltpu.VMEM((K, L), i32),
            pltpu.VMEM((K * L, d), f32),
            pltpu.VMEM((n_pad // NS, d), f32),
            pltpu.VMEM_SHARED((n_pad, d), f32),
            pltpu.SemaphoreType.DMA,
            pltpu.SemaphoreType.DMA,
        ],
    )
    def row_agg(src_hbm, dst_hbm, xs_hbm, zeros_hbm, out_hbm,
                sidx, didx, rows, bnc, acc, gsem, ssem):
        c = lax.axis_index("c")
        s = lax.axis_index("s")

        @pl.when(c == 0)
        def _():
            pltpu.sync_copy(xs_hbm.at[pl.ds(s * S, S)], bnc)

        @pl.when(c != 0)
        def _():
            pltpu.sync_copy(zeros_hbm.at[pl.ds(s * S, S)], bnc)

        pltpu.sync_copy(bnc, acc.at[pl.ds(s * S, S)])

        plsc.subcore_barrier()
        row_base = (c * NS + s) * jnp.int32(rows_per_sub)

        def body(g, carry):
            pltpu.sync_copy(src_hbm.at[pl.ds(row_base + g * jnp.int32(K), K)], sidx)
            pltpu.sync_copy(dst_hbm.at[pl.ds(row_base + g * jnp.int32(K), K)], didx)
            gs = [pltpu.async_copy(xs_hbm.at[sidx.at[jnp.int32(j)]],
                                   rows.at[pl.ds(jnp.int32(j * L), L)], gsem)
                  for j in range(K)]
            for g_ in gs:
                g_.wait()
            ss = [pltpu.async_copy(rows.at[pl.ds(jnp.int32(j * L), L)],
                                   acc.at[didx.at[jnp.int32(j)]], ssem, add=True)
                  for j in range(K)]
            for s_ in ss:
                s_.wait()
            return carry

        lax.fori_loop(jnp.int32(0), jnp.int32(rows_per_sub // K), body, jnp.int32(0))
        plsc.subcore_barrier()
        pltpu.sync_copy(acc.at[pl.ds(s * S, S)], bnc)
        pltpu.sync_copy(bnc, out_hbm.at[pl.ds(c * n_pad + s * S, S)])

    return row_agg


def _make_scalar_agg_kernel(n_pad, rows_per_sub):
    S = n_pad // NS

    @functools.partial(
        pl.kernel,
        out_type=jax.ShapeDtypeStruct((NC * n_pad,), f32),
        mesh=_mesh(),
        scratch_types=[
            pltpu.VMEM((K, L), i32),
            pltpu.VMEM((K, L), i32),
            pltpu.VMEM((K * L,), f32),
            pltpu.VMEM((n_pad // NS,), f32),
            pltpu.VMEM_SHARED((n_pad,), f32),
            pltpu.SemaphoreType.DMA,
            pltpu.SemaphoreType.DMA,
        ],
    )
    def scalar_agg(src_hbm, dst_hbm, ss_hbm, zeros_hbm, out_hbm,
                   sidx, didx, vals, bnc, acc, gsem, ssem):
        c = lax.axis_index("c")
        s = lax.axis_index("s")

        @pl.when(c == 0)
        def _():
            pltpu.sync_copy(ss_hbm.at[pl.ds(s * S, S)], bnc)

        @pl.when(c != 0)
        def _():
            pltpu.sync_copy(zeros_hbm.at[pl.ds(s * S, S)], bnc)

        pltpu.sync_copy(bnc, acc.at[pl.ds(s * S, S)])

        plsc.subcore_barrier()
        row_base = (c * NS + s) * jnp.int32(rows_per_sub)

        def body(g, carry):
            pltpu.sync_copy(src_hbm.at[pl.ds(row_base + g * jnp.int32(K), K)], sidx)
            pltpu.sync_copy(dst_hbm.at[pl.ds(row_base + g * jnp.int32(K), K)], didx)
            gs = [pltpu.async_copy(ss_hbm.at[sidx.at[jnp.int32(j)]],
                                   vals.at[pl.ds(jnp.int32(j * L), L)], gsem)
                  for j in range(K)]
            for g_ in gs:
                g_.wait()
            sc = [pltpu.async_copy(vals.at[pl.ds(jnp.int32(j * L), L)],
                                   acc.at[didx.at[jnp.int32(j)]], ssem, add=True)
                  for j in range(K)]
            for s_ in sc:
                s_.wait()
            return carry

        lax.fori_loop(jnp.int32(0), jnp.int32(rows_per_sub // K), body, jnp.int32(0))
        plsc.subcore_barrier()
        pltpu.sync_copy(acc.at[pl.ds(s * S, S)], bnc)
        pltpu.sync_copy(bnc, out_hbm.at[pl.ds(c * n_pad + s * S, S)])

    return scalar_agg


def _tc1(d0, d1, x_pad):
    def body(d0_ref, d1_ref, x_ref, dinv_ref, xs_ref):
        deg = d0_ref[...] + d1_ref[...]
        dinv = jnp.where(deg > 0.0, lax.rsqrt(deg), 0.0)
        dinv_ref[...] = dinv
        xs_ref[...] = x_ref[...] * dinv

    n_pad = x_pad.shape[0]
    d = x_pad.shape[1]
    nb = 8
    blk = n_pad // nb
    return pl.pallas_call(
        body,
        grid=(nb,),
        in_specs=[pl.BlockSpec((blk, 1), lambda i: (i, jnp.int32(0))),
                  pl.BlockSpec((blk, 1), lambda i: (i, jnp.int32(0))),
                  pl.BlockSpec((blk, d), lambda i: (i, jnp.int32(0)))],
        out_specs=[pl.BlockSpec((blk, 1), lambda i: (i, jnp.int32(0))),
                   pl.BlockSpec((blk, d), lambda i: (i, jnp.int32(0)))],
        out_shape=[jax.ShapeDtypeStruct((n_pad, 1), f32),
                   jax.ShapeDtypeStruct((n_pad, d), f32)],
    )(d0.reshape(n_pad, 1), d1.reshape(n_pad, 1), x_pad)


def _tc2(tp0, tp1, dinv, W1, b1, W2):
    def body(t0_ref, t1_ref, dinv_ref, w1_ref, b1_ref, w2_ref, ss_ref):
        t = t0_ref[...] + t1_ref[...]
        y1 = t * dinv_ref[...]
        h = lax.dot_general(y1, w1_ref[...], (((1,), (0,)), ((), ())),
                            preferred_element_type=f32) + b1_ref[...]
        h = jnp.maximum(h, 0.0)
        sv = lax.dot_general(h, w2_ref[...], (((1,), (0,)), ((), ())),
                             preferred_element_type=f32)
        ss_ref[...] = sv * dinv_ref[...]

    n_pad = tp0.shape[0]
    d = tp0.shape[1]
    h = W1.shape[1]
    nb = 8
    blk = n_pad // nb
    return pl.pallas_call(
        body,
        grid=(nb,),
        in_specs=[pl.BlockSpec((blk, d), lambda i: (i, jnp.int32(0))),
                  pl.BlockSpec((blk, d), lambda i: (i, jnp.int32(0))),
                  pl.BlockSpec((blk, 1), lambda i: (i, jnp.int32(0))),
                  pl.BlockSpec((d, h), lambda i: (jnp.int32(0), jnp.int32(0))),
                  pl.BlockSpec((1, h), lambda i: (jnp.int32(0), jnp.int32(0))),
                  pl.BlockSpec((h, 1), lambda i: (jnp.int32(0), jnp.int32(0)))],
        out_specs=pl.BlockSpec((blk, 1), lambda i: (i, jnp.int32(0))),
        out_shape=jax.ShapeDtypeStruct((n_pad, 1), f32),
    )(tp0, tp1, dinv, W1, b1.reshape(1, -1), W2)


def _tc3(t0, t1, dinv, b2):
    def body(t0_ref, t1_ref, dinv_ref, b2_ref, out_ref):
        out_ref[...] = (t0_ref[...] + t1_ref[...]) * dinv_ref[...] + b2_ref[...]

    n_pad = t0.shape[0]
    nb = 8
    blk = n_pad // nb
    return pl.pallas_call(
        body,
        grid=(nb,),
        in_specs=[pl.BlockSpec((blk, 1), lambda i: (i, jnp.int32(0))),
                  pl.BlockSpec((blk, 1), lambda i: (i, jnp.int32(0))),
                  pl.BlockSpec((blk, 1), lambda i: (i, jnp.int32(0))),
                  pl.BlockSpec((1, 1), lambda i: (jnp.int32(0), jnp.int32(0)))],
        out_specs=pl.BlockSpec((blk, 1), lambda i: (i, jnp.int32(0))),
        out_shape=jax.ShapeDtypeStruct((n_pad, 1), f32),
    )(t0.reshape(n_pad, 1), t1.reshape(n_pad, 1), dinv, b2.reshape(1, 1))


def kernel(x, edge_index, W1, b1, W2, b2):
    n = x.shape[0]
    d = x.shape[1]
    e = edge_index.shape[1]

    n_pad = _ru(n + 1, NS * 8)
    S = n_pad // NS
    del S
    rows = _ru((e + L - 1) // L, NC * NS * K) // (NC * NS)
    e_pad = rows * NC * NS * L

    src = edge_index[0].astype(i32)
    dst = edge_index[1].astype(i32)
    pad_cfg = [(0, e_pad - e)]
    src2d = jnp.pad(src, pad_cfg, constant_values=n).reshape(NC * NS * rows, L)
    dst2d = jnp.pad(dst, pad_cfg, constant_values=n).reshape(NC * NS * rows, L)

    x_pad = jnp.pad(x.astype(f32), [(0, n_pad - n), (0, 0)])
    ones_n = (jnp.arange(n_pad, dtype=i32) < n).astype(f32)
    zeros1 = jnp.zeros((n_pad,), f32)
    zeros2 = jnp.zeros((n_pad, d), f32)

    degp = _make_deg_kernel(n_pad, rows)(dst2d, ones_n, zeros1)
    dinv, xs = _tc1(degp[:n_pad], degp[n_pad:], x_pad)

    tp = _make_row_agg_kernel(n_pad, rows, d)(src2d, dst2d, xs, zeros2)
    ss = _tc2(tp[:n_pad], tp[n_pad:], dinv, W1.astype(f32), b1.astype(f32),
              W2.astype(f32))

    ssf = ss.reshape(n_pad)
    t2p = _make_scalar_agg_kernel(n_pad, rows)(src2d, dst2d, ssf, zeros1)
    out = _tc3(t2p[:n_pad], t2p[n_pad:], dinv, b2.astype(f32))
    return out[:n, 0]


# speedup vs baseline: 114.5765x; 1.3012x over previous
import functools

import jax
import jax.numpy as jnp
from jax import lax
from jax.experimental import pallas as pl
from jax.experimental.pallas import tpu as pltpu
from jax.experimental.pallas import tpu_sc as plsc

NC = 2
NS = 16
L = 128
K = 8

f32 = jnp.float32
i32 = jnp.int32


def _ru(a, b):
    return (a + b - 1) // b * b


def _mesh():
    return plsc.VectorSubcoreMesh(core_axis_name="c", subcore_axis_name="s")


def _i(v):
    return jnp.int32(v)


def _make_deg_kernel(n_pad, rows_per_sub):
    S = n_pad // NS
    G = rows_per_sub // K
    assert G % 2 == 0

    @functools.partial(
        pl.kernel,
        out_type=jax.ShapeDtypeStruct((NC * n_pad,), f32),
        mesh=_mesh(),
        scratch_types=[
            pltpu.VMEM((K, L), i32),
            pltpu.VMEM((K, L), i32),
            pltpu.VMEM((L,), f32),
            pltpu.VMEM((n_pad // NS,), f32),
            pltpu.VMEM_SHARED((n_pad,), f32),
            pltpu.SemaphoreType.DMA,
            pltpu.SemaphoreType.DMA,
        ],
    )
    def deg_kernel(dst_hbm, ones_hbm, zeros_hbm, out_hbm,
                   didx0, didx1, ones_v, bnc, acc, ssem0, ssem1):
        c = lax.axis_index("c")
        s = lax.axis_index("s")
        didx = (didx0, didx1)
        ssem = (ssem0, ssem1)
        pltpu.sync_copy(ones_hbm.at[pl.ds(0, L)], ones_v)

        @pl.when(c == 0)
        def _():
            pltpu.sync_copy(ones_hbm.at[pl.ds(s * S, S)], bnc)

        @pl.when(c != 0)
        def _():
            pltpu.sync_copy(zeros_hbm.at[pl.ds(s * S, S)], bnc)

        pltpu.sync_copy(bnc, acc.at[pl.ds(s * S, S)])
        plsc.subcore_barrier()
        row_base = (c * NS + s) * _i(rows_per_sub)

        def wait_s(b):
            for j in range(K):
                pltpu.make_async_copy(
                    ones_v, acc.at[didx[b].at[_i(j)]], ssem[b]).wait()

        def half(g, b):
            @pl.when(g >= 2)
            def _():
                wait_s(b)
            pltpu.sync_copy(dst_hbm.at[pl.ds(row_base + g * _i(K), K)],
                            didx[b])
            for j in range(K):
                pltpu.async_copy(ones_v, acc.at[didx[b].at[_i(j)]],
                                 ssem[b], add=True)

        def body(g0, carry):
            half(g0 * _i(2), 0)
            half(g0 * _i(2) + _i(1), 1)
            return carry

        lax.fori_loop(_i(0), _i(G // 2), body, _i(0))
        wait_s(0)
        wait_s(1)
        plsc.subcore_barrier()
        pltpu.sync_copy(acc.at[pl.ds(s * S, S)], bnc)
        pltpu.sync_copy(bnc, out_hbm.at[pl.ds(c * n_pad + s * S, S)])

    return deg_kernel


def _make_agg_kernel(n_pad, rows_per_sub, d):
    S = n_pad // NS
    G = rows_per_sub // K
    assert G % 2 == 0

    def shp(m):
        return (m,) if d == 1 else (m, d)

    @functools.partial(
        pl.kernel,
        out_type=jax.ShapeDtypeStruct(shp(NC * n_pad), f32),
        mesh=_mesh(),
        compiler_params=pltpu.CompilerParams(use_tc_tiling_on_sc=False),
        scratch_types=[
            pltpu.VMEM((K, L), i32),
            pltpu.VMEM((K, L), i32),
            pltpu.VMEM((K, L), i32),
            pltpu.VMEM((K, L), i32),
            pltpu.VMEM(shp(K * L), f32),
            pltpu.VMEM(shp(K * L), f32),
            pltpu.VMEM_SHARED(shp(n_pad), f32),
            pltpu.SemaphoreType.DMA,
            pltpu.SemaphoreType.DMA,
            pltpu.SemaphoreType.DMA,
            pltpu.SemaphoreType.DMA,
        ],
    )
    def agg(src_hbm, dst_hbm, xs_hbm, zeros_hbm, out_hbm,
            sidx0, sidx1, didx0, didx1, rows0, rows1, acc,
            gsem0, gsem1, ssem0, ssem1):
        c = lax.axis_index("c")
        s = lax.axis_index("s")
        sidx = (sidx0, sidx1)
        didx = (didx0, didx1)
        rows = (rows0, rows1)
        gsem = (gsem0, gsem1)
        ssem = (ssem0, ssem1)

        CH = K * L
        chunks = [(o, min(CH, S - o)) for o in range(0, S, CH)]

        def staged_copy(src_ref, src_off, dst_ref, dst_off):
            for o, w in chunks:
                pltpu.sync_copy(src_ref.at[pl.ds(src_off + _i(o), w)],
                                rows0.at[pl.ds(_i(0), w)])
                pltpu.sync_copy(rows0.at[pl.ds(_i(0), w)],
                                dst_ref.at[pl.ds(dst_off + _i(o), w)])

        @pl.when(c == 0)
        def _():
            staged_copy(xs_hbm, s * S, acc, s * S)

        @pl.when(c != 0)
        def _():
            staged_copy(zeros_hbm, s * S, acc, s * S)

        plsc.subcore_barrier()
        row_base = (c * NS + s) * _i(rows_per_sub)

        def load_and_gather(g, b):
            pltpu.sync_copy(src_hbm.at[pl.ds(row_base + g * _i(K), K)],
                            sidx[b])
            pltpu.sync_copy(dst_hbm.at[pl.ds(row_base + g * _i(K), K)],
                            didx[b])
            for j in range(K):
                pltpu.async_copy(xs_hbm.at[sidx[b].at[_i(j)]],
                                 rows[b].at[pl.ds(_i(j * L), L)], gsem[b])

        def wait_s(b):
            for j in range(K):
                pltpu.make_async_copy(rows[b].at[pl.ds(_i(j * L), L)],
                                      acc.at[didx[b].at[_i(j)]], ssem[b]).wait()

        def half(g, b, nb):
            @pl.when(g >= 1)
            def _():
                wait_s(nb)

            @pl.when(g <= G - 2)
            def _():
                load_and_gather(g + _i(1), nb)

            for j in range(K):
                pltpu.make_async_copy(xs_hbm.at[sidx[b].at[_i(j)]],
                                      rows[b].at[pl.ds(_i(j * L), L)],
                                      gsem[b]).wait()
            for j in range(K):
                pltpu.async_copy(rows[b].at[pl.ds(_i(j * L), L)],
                                 acc.at[didx[b].at[_i(j)]], ssem[b], add=True)

        def body(g0, carry):
            half(g0 * _i(2), 0, 1)
            half(g0 * _i(2) + _i(1), 1, 0)
            return carry

        load_and_gather(_i(0), 0)
        lax.fori_loop(_i(0), _i(G // 2), body, _i(0))
        wait_s(1)
        plsc.subcore_barrier()
        staged_copy(acc, s * S, out_hbm, c * n_pad + s * S)

    return agg


def _tc1(d0, d1, x_pad):
    def body(d0_ref, d1_ref, x_ref, dinv_ref, xs_ref):
        deg = d0_ref[...] + d1_ref[...]
        dinv = jnp.where(deg > 0.0, lax.rsqrt(deg), 0.0)
        dinv_ref[...] = dinv
        xs_ref[...] = x_ref[...] * dinv

    n_pad = x_pad.shape[0]
    d = x_pad.shape[1]
    nb = 8
    blk = n_pad // nb
    return pl.pallas_call(
        body,
        grid=(nb,),
        in_specs=[pl.BlockSpec((blk, 1), lambda i: (i, _i(0))),
                  pl.BlockSpec((blk, 1), lambda i: (i, _i(0))),
                  pl.BlockSpec((blk, d), lambda i: (i, _i(0)))],
        out_specs=[pl.BlockSpec((blk, 1), lambda i: (i, _i(0))),
                   pl.BlockSpec((blk, d), lambda i: (i, _i(0)))],
        out_shape=[jax.ShapeDtypeStruct((n_pad, 1), f32),
                   jax.ShapeDtypeStruct((n_pad, d), f32)],
    )(d0.reshape(n_pad, 1), d1.reshape(n_pad, 1), x_pad)


def _tc2(tp0, tp1, dinv, W1, b1, W2):
    def body(t0_ref, t1_ref, dinv_ref, w1_ref, b1_ref, w2_ref, ss_ref):
        t = t0_ref[...] + t1_ref[...]
        y1 = t * dinv_ref[...]
        h = lax.dot_general(y1, w1_ref[...], (((1,), (0,)), ((), ())),
                            preferred_element_type=f32) + b1_ref[...]
        h = jnp.maximum(h, 0.0)
        sv = lax.dot_general(h, w2_ref[...], (((1,), (0,)), ((), ())),
                             preferred_element_type=f32)
        ss_ref[...] = sv * dinv_ref[...]

    n_pad = tp0.shape[0]
    d = tp0.shape[1]
    h = W1.shape[1]
    nb = 8
    blk = n_pad // nb
    return pl.pallas_call(
        body,
        grid=(nb,),
        in_specs=[pl.BlockSpec((blk, d), lambda i: (i, _i(0))),
                  pl.BlockSpec((blk, d), lambda i: (i, _i(0))),
                  pl.BlockSpec((blk, 1), lambda i: (i, _i(0))),
                  pl.BlockSpec((d, h), lambda i: (_i(0), _i(0))),
                  pl.BlockSpec((1, h), lambda i: (_i(0), _i(0))),
                  pl.BlockSpec((h, 1), lambda i: (_i(0), _i(0)))],
        out_specs=pl.BlockSpec((blk, 1), lambda i: (i, _i(0))),
        out_shape=jax.ShapeDtypeStruct((n_pad, 1), f32),
    )(tp0, tp1, dinv, W1, b1.reshape(1, -1), W2)


def _tc3(t0, t1, dinv, b2):
    def body(t0_ref, t1_ref, dinv_ref, b2_ref, out_ref):
        out_ref[...] = (t0_ref[...] + t1_ref[...]) * dinv_ref[...] + b2_ref[...]

    n_pad = t0.shape[0]
    nb = 8
    blk = n_pad // nb
    return pl.pallas_call(
        body,
        grid=(nb,),
        in_specs=[pl.BlockSpec((blk, 1), lambda i: (i, _i(0))),
                  pl.BlockSpec((blk, 1), lambda i: (i, _i(0))),
                  pl.BlockSpec((blk, 1), lambda i: (i, _i(0))),
                  pl.BlockSpec((1, 1), lambda i: (_i(0), _i(0)))],
        out_specs=pl.BlockSpec((blk, 1), lambda i: (i, _i(0))),
        out_shape=jax.ShapeDtypeStruct((n_pad, 1), f32),
    )(t0, t1, dinv, b2.reshape(1, 1))


def kernel(x, edge_index, W1, b1, W2, b2):
    n = x.shape[0]
    d = x.shape[1]
    e = edge_index.shape[1]

    n_pad = _ru(n + 1, NS * 8)
    rows = _ru((e + L - 1) // L, NC * NS * 2 * K) // (NC * NS)
    e_pad = rows * NC * NS * L

    src = edge_index[0].astype(i32)
    dst = edge_index[1].astype(i32)
    pad_cfg = [(0, e_pad - e)]
    src2d = jnp.pad(src, pad_cfg, constant_values=n).reshape(NC * NS * rows, L)
    dst2d = jnp.pad(dst, pad_cfg, constant_values=n).reshape(NC * NS * rows, L)

    x_pad = jnp.pad(x.astype(f32), [(0, n_pad - n), (0, 0)])
    ones_n = (jnp.arange(n_pad, dtype=i32) < n).astype(f32)
    zeros1 = jnp.zeros((n_pad,), f32)
    zeros2 = jnp.zeros((n_pad, d), f32)

    degp = _make_deg_kernel(n_pad, rows)(dst2d, ones_n, zeros1)
    dinv, xs = _tc1(degp[:n_pad], degp[n_pad:], x_pad)

    tp = _make_agg_kernel(n_pad, rows, d)(src2d, dst2d, xs, zeros2)
    ss = _tc2(tp[:n_pad], tp[n_pad:], dinv, W1.astype(f32), b1.astype(f32),
              W2.astype(f32))

    ssf = ss.reshape(n_pad)
    t2p = _make_agg_kernel(n_pad, rows, 1)(src2d, dst2d, ssf, zeros1)
    t2p = t2p.reshape(NC * n_pad, 1)
    out = _tc3(t2p[:n_pad], t2p[n_pad:], dinv, b2.astype(f32))
    return out[:n, 0]


# speedup vs baseline: 136.2291x; 1.1890x over previous
import functools

import jax
import jax.numpy as jnp
from jax import lax
from jax.experimental import pallas as pl
from jax.experimental.pallas import tpu as pltpu
from jax.experimental.pallas import tpu_sc as plsc

NC = 2
NS = 16
L = 128
K = 8

f32 = jnp.float32
i32 = jnp.int32


def _ru(a, b):
    return (a + b - 1) // b * b


def _mesh():
    return plsc.VectorSubcoreMesh(core_axis_name="c", subcore_axis_name="s")


def _i(v):
    return jnp.int32(v)


def _make_deg_kernel(n_pad, rows_per_sub):
    S = n_pad // NS
    G = rows_per_sub // K
    assert G % 2 == 0

    @functools.partial(
        pl.kernel,
        out_type=jax.ShapeDtypeStruct((NC * n_pad,), f32),
        mesh=_mesh(),
        scratch_types=[
            pltpu.VMEM((K, L), i32),
            pltpu.VMEM((K, L), i32),
            pltpu.VMEM((L,), f32),
            pltpu.VMEM((n_pad // NS,), f32),
            pltpu.VMEM_SHARED((n_pad,), f32),
            pltpu.SemaphoreType.DMA,
            pltpu.SemaphoreType.DMA,
        ],
    )
    def deg_kernel(dst_hbm, ones_hbm, zeros_hbm, out_hbm,
                   didx0, didx1, ones_v, bnc, acc, ssem0, ssem1):
        c = lax.axis_index("c")
        s = lax.axis_index("s")
        didx = (didx0, didx1)
        ssem = (ssem0, ssem1)
        pltpu.sync_copy(ones_hbm.at[pl.ds(0, L)], ones_v)

        @pl.when(c == 0)
        def _():
            pltpu.sync_copy(ones_hbm.at[pl.ds(s * S, S)], bnc)

        @pl.when(c != 0)
        def _():
            pltpu.sync_copy(zeros_hbm.at[pl.ds(s * S, S)], bnc)

        pltpu.sync_copy(bnc, acc.at[pl.ds(s * S, S)])
        plsc.subcore_barrier()
        row_base = (c * NS + s) * _i(rows_per_sub)

        def wait_s(b):
            for j in range(K):
                pltpu.make_async_copy(
                    ones_v, acc.at[didx[b].at[_i(j)]], ssem[b]).wait()

        def half(g, b):
            @pl.when(g >= 2)
            def _():
                wait_s(b)
            pltpu.sync_copy(dst_hbm.at[pl.ds(row_base + g * _i(K), K)],
                            didx[b])
            for j in range(K):
                pltpu.async_copy(ones_v, acc.at[didx[b].at[_i(j)]],
                                 ssem[b], add=True)

        def body(g0, carry):
            half(g0 * _i(2), 0)
            half(g0 * _i(2) + _i(1), 1)
            return carry

        lax.fori_loop(_i(0), _i(G // 2), body, _i(0))
        wait_s(0)
        wait_s(1)
        plsc.subcore_barrier()
        pltpu.sync_copy(acc.at[pl.ds(s * S, S)], bnc)
        pltpu.sync_copy(bnc, out_hbm.at[pl.ds(c * n_pad + s * S, S)])

    return deg_kernel


def _make_agg_kernel(n_pad, rows_per_sub, d):
    S = n_pad // NS
    G = rows_per_sub // K
    assert G % 2 == 0

    def shp(m):
        return (m,) if d == 1 else (m, d)

    @functools.partial(
        pl.kernel,
        out_type=jax.ShapeDtypeStruct(shp(NC * n_pad), f32),
        mesh=_mesh(),
        compiler_params=pltpu.CompilerParams(use_tc_tiling_on_sc=False),
        scratch_types=[
            pltpu.VMEM((K, L), i32),
            pltpu.VMEM((K, L), i32),
            pltpu.VMEM((K, L), i32),
            pltpu.VMEM((K, L), i32),
            pltpu.VMEM(shp(K * L), f32),
            pltpu.VMEM(shp(K * L), f32),
            pltpu.VMEM_SHARED(shp(n_pad), f32),
            pltpu.SemaphoreType.DMA,
            pltpu.SemaphoreType.DMA,
            pltpu.SemaphoreType.DMA,
            pltpu.SemaphoreType.DMA,
        ],
    )
    def agg(src_hbm, dst_hbm, xs_hbm, zeros_hbm, out_hbm,
            sidx0, sidx1, didx0, didx1, rows0, rows1, acc,
            gsem0, gsem1, ssem0, ssem1):
        c = lax.axis_index("c")
        s = lax.axis_index("s")
        sidx = (sidx0, sidx1)
        didx = (didx0, didx1)
        rows = (rows0, rows1)
        gsem = (gsem0, gsem1)
        ssem = (ssem0, ssem1)

        CH = K * L
        chunks = [(o, min(CH, S - o)) for o in range(0, S, CH)]

        def staged_copy(src_ref, src_off, dst_ref, dst_off):
            for o, w in chunks:
                pltpu.sync_copy(src_ref.at[pl.ds(src_off + _i(o), w)],
                                rows0.at[pl.ds(_i(0), w)])
                pltpu.sync_copy(rows0.at[pl.ds(_i(0), w)],
                                dst_ref.at[pl.ds(dst_off + _i(o), w)])

        @pl.when(c == 0)
        def _():
            staged_copy(xs_hbm, s * S, acc, s * S)

        @pl.when(c != 0)
        def _():
            staged_copy(zeros_hbm, s * S, acc, s * S)

        plsc.subcore_barrier()
        row_base = (c * NS + s) * _i(rows_per_sub)

        def load_and_gather(g, b):
            pltpu.sync_copy(src_hbm.at[pl.ds(row_base + g * _i(K), K)],
                            sidx[b])
            pltpu.sync_copy(dst_hbm.at[pl.ds(row_base + g * _i(K), K)],
                            didx[b])
            for j in range(K):
                pltpu.async_copy(xs_hbm.at[sidx[b].at[_i(j)]],
                                 rows[b].at[pl.ds(_i(j * L), L)], gsem[b])

        def wait_s(b):
            for j in range(K):
                pltpu.make_async_copy(rows[b].at[pl.ds(_i(j * L), L)],
                                      acc.at[didx[b].at[_i(j)]], ssem[b]).wait()

        def half(g, b, nb):
            @pl.when(g >= 1)
            def _():
                wait_s(nb)

            @pl.when(g <= G - 2)
            def _():
                load_and_gather(g + _i(1), nb)

            for j in range(K):
                pltpu.make_async_copy(xs_hbm.at[sidx[b].at[_i(j)]],
                                      rows[b].at[pl.ds(_i(j * L), L)],
                                      gsem[b]).wait()
            for j in range(K):
                pltpu.async_copy(rows[b].at[pl.ds(_i(j * L), L)],
                                 acc.at[didx[b].at[_i(j)]], ssem[b], add=True)

        def body(g0, carry):
            half(g0 * _i(2), 0, 1)
            half(g0 * _i(2) + _i(1), 1, 0)
            return carry

        load_and_gather(_i(0), 0)
        lax.fori_loop(_i(0), _i(G // 2), body, _i(0))
        wait_s(1)
        plsc.subcore_barrier()
        staged_copy(acc, s * S, out_hbm, c * n_pad + s * S)

    return agg


def _tc1(d0, d1, x_pad):
    def body(d0_ref, d1_ref, x_ref, dinv_ref, xs_ref):
        deg = d0_ref[...] + d1_ref[...]
        dinv = jnp.where(deg > 0.0, lax.rsqrt(deg), 0.0)
        dinv_ref[...] = dinv
        xs_ref[...] = x_ref[...] * dinv

    n_pad = x_pad.shape[0]
    d = x_pad.shape[1]
    nb = 8
    blk = n_pad // nb
    return pl.pallas_call(
        body,
        grid=(nb,),
        in_specs=[pl.BlockSpec((blk, 1), lambda i: (i, _i(0))),
                  pl.BlockSpec((blk, 1), lambda i: (i, _i(0))),
                  pl.BlockSpec((blk, d), lambda i: (i, _i(0)))],
        out_specs=[pl.BlockSpec((blk, 1), lambda i: (i, _i(0))),
                   pl.BlockSpec((blk, d), lambda i: (i, _i(0)))],
        out_shape=[jax.ShapeDtypeStruct((n_pad, 1), f32),
                   jax.ShapeDtypeStruct((n_pad, d), f32)],
    )(d0.reshape(n_pad, 1), d1.reshape(n_pad, 1), x_pad)


def _tc2(tp0, tp1, dinv, W1, b1, W2):
    def body(t0_ref, t1_ref, dinv_ref, w1_ref, b1_ref, w2_ref, ss_ref):
        t = t0_ref[...] + t1_ref[...]
        y1 = t * dinv_ref[...]
        h = lax.dot_general(y1, w1_ref[...], (((1,), (0,)), ((), ())),
                            preferred_element_type=f32) + b1_ref[...]
        h = jnp.maximum(h, 0.0)
        sv = lax.dot_general(h, w2_ref[...], (((1,), (0,)), ((), ())),
                             preferred_element_type=f32)
        ss_ref[...] = sv * dinv_ref[...]

    n_pad = tp0.shape[0]
    d = tp0.shape[1]
    h = W1.shape[1]
    nb = 8
    blk = n_pad // nb
    return pl.pallas_call(
        body,
        grid=(nb,),
        in_specs=[pl.BlockSpec((blk, d), lambda i: (i, _i(0))),
                  pl.BlockSpec((blk, d), lambda i: (i, _i(0))),
                  pl.BlockSpec((blk, 1), lambda i: (i, _i(0))),
                  pl.BlockSpec((d, h), lambda i: (_i(0), _i(0))),
                  pl.BlockSpec((1, h), lambda i: (_i(0), _i(0))),
                  pl.BlockSpec((h, 1), lambda i: (_i(0), _i(0)))],
        out_specs=pl.BlockSpec((blk, 1), lambda i: (i, _i(0))),
        out_shape=jax.ShapeDtypeStruct((n_pad, 1), f32),
    )(tp0, tp1, dinv, W1, b1.reshape(1, -1), W2)


def _tc3(t0, t1, dinv, b2):
    def body(t0_ref, t1_ref, dinv_ref, b2_ref, out_ref):
        out_ref[...] = (t0_ref[...] + t1_ref[...]) * dinv_ref[...] + b2_ref[...]

    n_pad = t0.shape[0]
    nb = 8
    blk = n_pad // nb
    return pl.pallas_call(
        body,
        grid=(nb,),
        in_specs=[pl.BlockSpec((blk, 1), lambda i: (i, _i(0))),
                  pl.BlockSpec((blk, 1), lambda i: (i, _i(0))),
                  pl.BlockSpec((blk, 1), lambda i: (i, _i(0))),
                  pl.BlockSpec((1, 1), lambda i: (_i(0), _i(0)))],
        out_specs=pl.BlockSpec((blk, 1), lambda i: (i, _i(0))),
        out_shape=jax.ShapeDtypeStruct((n_pad, 1), f32),
    )(t0, t1, dinv, b2.reshape(1, 1))


def _make_scalar_cp_kernel(n_pad, rows_per_sub):
    RB = n_pad // 16
    RBp = _ru(RB, L)
    SW = RBp // NS
    G = rows_per_sub // K
    CL = K * L
    assert G % 2 == 0 and RB % NS == 0

    @functools.partial(
        pl.kernel,
        out_type=jax.ShapeDtypeStruct((NC * RBp, 16), f32),
        mesh=_mesh(),
        compiler_params=pltpu.CompilerParams(use_tc_tiling_on_sc=False,
                                             needs_layout_passes=False),
        scratch_types=[
            pltpu.VMEM((CL,), i32),
            pltpu.VMEM((CL,), i32),
            pltpu.VMEM((CL,), i32),
            pltpu.VMEM((CL,), i32),
            pltpu.VMEM((RBp, 16), f32),
            pltpu.VMEM((RBp, 16), f32),
            pltpu.VMEM((RBp // L, L), i32),
            pltpu.VMEM_SHARED((RBp, 16), f32),
            pltpu.SemaphoreType.DMA,
            pltpu.SemaphoreType.DMA,
            pltpu.SemaphoreType.DMA,
        ],
    )
    def scal(srcf_hbm, dstf_hbm, ss_hbm, zeros_hbm, iota_hbm, out_hbm,
             sidx0, sidx1, didx0, didx1, ss_v, acc_v, iota_v, accs,
             isem0, isem1, rsem):
        c = lax.axis_index("c")
        s = lax.axis_index("s")
        sidx = (sidx0, sidx1)
        didx = (didx0, didx1)
        isem = (isem0, isem1)

        pltpu.sync_copy(iota_hbm, iota_v)
        pltpu.sync_copy(ss_hbm, ss_v)

        @pl.when(c == 0)
        def _():
            pltpu.sync_copy(ss_hbm.at[pl.ds(s * SW, SW)],
                            acc_v.at[pl.ds(_i(0), SW)])

        @pl.when(c != 0)
        def _():
            pltpu.sync_copy(zeros_hbm.at[pl.ds(s * SW, SW)],
                            acc_v.at[pl.ds(_i(0), SW)])

        pltpu.sync_copy(acc_v.at[pl.ds(_i(0), SW)], accs.at[pl.ds(s * SW, SW)])
        pltpu.sync_copy(zeros_hbm, acc_v)
        plsc.subcore_barrier()

        ebase = (c * NS + s) * _i(rows_per_sub * L)

        def load_idx(g, b):
            return [pltpu.async_copy(
                        srcf_hbm.at[pl.ds(ebase + g * _i(CL), CL)],
                        sidx[b], isem[b]),
                    pltpu.async_copy(
                        dstf_hbm.at[pl.ds(ebase + g * _i(CL), CL)],
                        didx[b], isem[b])]

        def wait_idx(g, b):
            pltpu.make_async_copy(srcf_hbm.at[pl.ds(ebase + g * _i(CL), CL)],
                                  sidx[b], isem[b]).wait()
            pltpu.make_async_copy(dstf_hbm.at[pl.ds(ebase + g * _i(CL), CL)],
                                  didx[b], isem[b]).wait()

        def compute(b):
            for m in range(CL // 16):
                sv = sidx[b][pl.ds(_i(m * 16), 16)]
                dv = didx[b][pl.ds(_i(m * 16), 16)]
                vals = plsc.load_gather(
                    ss_v, [jnp.right_shift(sv, _i(4)),
                           jnp.bitwise_and(sv, _i(15))])
                plsc.addupdate_scatter(
                    acc_v, [jnp.right_shift(dv, _i(4)),
                            jnp.bitwise_and(dv, _i(15))], vals)

        def half(g, b, nb):
            @pl.when(g >= 1)
            def _():
                wait_idx(g, b)

            @pl.when(g <= G - 2)
            def _():
                load_idx(g + _i(1), nb)

            compute(b)

        def body(g0, carry):
            half(g0 * _i(2), 0, 1)
            half(g0 * _i(2) + _i(1), 1, 0)
            return carry

        pltpu.sync_copy(srcf_hbm.at[pl.ds(ebase, CL)], sidx[0])
        pltpu.sync_copy(dstf_hbm.at[pl.ds(ebase, CL)], didx[0])
        lax.fori_loop(_i(0), _i(G // 2), body, _i(0))

        rds = [pltpu.async_copy(acc_v.at[pl.ds(_i(k * L), L)],
                                accs.at[iota_v.at[_i(k)]], rsem, add=True)
               for k in range(RBp // L)]
        for r_ in rds:
            r_.wait()
        plsc.subcore_barrier()
        pltpu.sync_copy(accs.at[pl.ds(s * SW, SW)],
                        acc_v.at[pl.ds(_i(0), SW)])
        pltpu.sync_copy(acc_v.at[pl.ds(_i(0), SW)],
                        out_hbm.at[pl.ds(c * RBp + s * SW, SW)])

    return scal


def kernel(x, edge_index, W1, b1, W2, b2):
    n = x.shape[0]
    d = x.shape[1]
    e = edge_index.shape[1]

    n_pad = _ru(n + 1, 256)
    rows = _ru((e + L - 1) // L, NC * NS * 2 * K) // (NC * NS)
    e_pad = rows * NC * NS * L

    src = edge_index[0].astype(i32)
    dst = edge_index[1].astype(i32)
    pad_cfg = [(0, e_pad - e)]
    src2d = jnp.pad(src, pad_cfg, constant_values=n).reshape(NC * NS * rows, L)
    dst2d = jnp.pad(dst, pad_cfg, constant_values=n).reshape(NC * NS * rows, L)

    x_pad = jnp.pad(x.astype(f32), [(0, n_pad - n), (0, 0)])
    ones_n = (jnp.arange(n_pad, dtype=i32) < n).astype(f32)
    zeros1 = jnp.zeros((n_pad,), f32)
    zeros2 = jnp.zeros((n_pad, d), f32)

    degp = _make_deg_kernel(n_pad, rows)(dst2d, ones_n, zeros1)
    dinv, xs = _tc1(degp[:n_pad], degp[n_pad:], x_pad)

    tp = _make_agg_kernel(n_pad, rows, d)(src2d, dst2d, xs, zeros2)
    ss = _tc2(tp[:n_pad], tp[n_pad:], dinv, W1.astype(f32), b1.astype(f32),
              W2.astype(f32))

    RB = n_pad // 16
    RBp = _ru(RB, L)
    ss2d = jnp.pad(ss.reshape(RB, 16), [(0, RBp - RB), (0, 0)])
    zeros_rbp = jnp.zeros((RBp, 16), f32)
    iota2d = jnp.arange(RBp, dtype=i32).reshape(RBp // L, L)
    t2p = _make_scalar_cp_kernel(n_pad, rows)(
        src2d.reshape(e_pad), dst2d.reshape(e_pad), ss2d, zeros_rbp, iota2d)
    t2 = t2p.reshape(NC, RBp, 16)[:, :RB, :].reshape(NC, n_pad)
    out = _tc3(t2[0].reshape(n_pad, 1), t2[1].reshape(n_pad, 1), dinv,
               b2.astype(f32))
    return out[:n, 0]
